# baseline (speedup 1.0000x reference)
import functools

import jax
import jax.numpy as jnp
from jax import lax
from jax.experimental import pallas as pl
from jax.experimental.pallas import tpu as pltpu
from jax.experimental.pallas import tpu_sc as plsc

EPS = 1e-6
THETA = 1000000.0
H = 16
HD = 128
T = 2048
D = 2048
I = 5504
BT = 256
NT = T // BT
WI = 256
NJ2 = (I + WI - 1) // WI
BTM = 512
NTM = T // BTM + 1
TP = NTM * BTM
SC_NC = 2
SC_NS = 16
SC_NW = SC_NC * SC_NS
SCALE = 1.0 / (HD ** 0.5)
NEG = -1e30
F32 = jnp.float32
BF16 = jnp.bfloat16


def _silu(x):
    return x / (1.0 + jnp.exp(-x))


def _addnorm1_body(h_ref, r_ref, w_ref, z_ref, xn_ref):
    z = h_ref[...] + r_ref[...]
    z_ref[...] = z
    v = jnp.mean(z * z, axis=1, keepdims=True)
    xn_ref[...] = (z * jax.lax.rsqrt(v + EPS) * w_ref[...]).astype(BF16)


def _addnorm1(h, r, ln1_row):
    return pl.pallas_call(
        _addnorm1_body,
        grid=(NT,),
        in_specs=[
            pl.BlockSpec((BT, D), lambda t: (t, 0)),
            pl.BlockSpec((BT, D), lambda t: (t, 0)),
            pl.BlockSpec((1, D), lambda t: (0, 0)),
        ],
        out_specs=[
            pl.BlockSpec((BT, D), lambda t: (t, 0)),
            pl.BlockSpec((BT, D), lambda t: (t, 0)),
        ],
        out_shape=[
            jax.ShapeDtypeStruct((T, D), F32),
            jax.ShapeDtypeStruct((T, D), BF16),
        ],
    )(h, r, ln1_row)


HG = 2
WQ = HG * HD


def _qkv_body(xn_ref, pos_ref, qw_ref, kw_ref, vw_ref, qb_ref, kb_ref, vb_ref,
              q_ref, k_ref, v_ref, cos_ref, sin_ref):
    j = pl.program_id(0)

    @pl.when(j == 0)
    def _():
        li = jax.lax.broadcasted_iota(jnp.int32, (T, WQ), 1)
        lh = li % HD
        f = jnp.where(lh < HD // 2, lh, lh - HD // 2).astype(F32)
        inv = jnp.exp(f * (-jnp.log(THETA) / (HD // 2)))
        freqs = pos_ref[...] * inv
        cos_ref[...] = jnp.cos(freqs)
        sgn = jnp.where(lh < HD // 2, -1.0, 1.0)
        sin_ref[...] = jnp.sin(freqs) * sgn

    xn = xn_ref[...]

    def proj(w_ref, b_ref):
        w = w_ref[...].astype(BF16)
        y = jnp.dot(xn, w, preferred_element_type=F32)
        return y + b_ref[0]

    def rope(x):
        x3 = x.reshape(T, HG, HD)
        xr = jnp.concatenate([x3[..., HD // 2:], x3[..., :HD // 2]],
                             axis=-1).reshape(T, WQ)
        return x * cos_ref[...] + xr * sin_ref[...]

    q_ref[...] = rope(proj(qw_ref, qb_ref)).astype(BF16)
    k_ref[...] = rope(proj(kw_ref, kb_ref)).astype(BF16)
    v_ref[...] = proj(vw_ref, vb_ref).astype(BF16)


def _qkv(xn, pos_col, q_w, k_w, v_w, qb3, kb3, vb3):
    wspec = pl.BlockSpec((D, WQ), lambda j: (0, j))
    bspec = pl.BlockSpec((1, 1, WQ), lambda j: (j, 0, 0))
    ospec = pl.BlockSpec((T, WQ), lambda j: (0, j))
    return pl.pallas_call(
        _qkv_body,
        grid=(H // HG,),
        in_specs=[
            pl.BlockSpec((T, D), lambda j: (0, 0)),
            pl.BlockSpec((T, 1), lambda j: (0, 0)),
            wspec, wspec, wspec, bspec, bspec, bspec,
        ],
        out_specs=[ospec, ospec, ospec],
        out_shape=[jax.ShapeDtypeStruct((T, D), BF16)] * 3,
        scratch_shapes=[
            pltpu.VMEM((T, WQ), F32),
            pltpu.VMEM((T, WQ), F32),
        ],
        compiler_params=pltpu.CompilerParams(vmem_limit_bytes=100 * 2**20),
    )(xn, pos_col, q_w, k_w, v_w, qb3, kb3, vb3)


BQ = 512
BK = 512
NQT = T // BQ
NKT = T // BK


def _attn_body(q_ref, k_ref, v_ref, o_ref, acc_ref, mx_ref, ls_ref):
    qt = pl.program_id(1)
    kt = pl.program_id(2)

    @pl.when(kt == 0)
    def _():
        mx_ref[...] = jnp.full((BQ, 1), NEG, F32)
        ls_ref[...] = jnp.zeros((BQ, 1), F32)
        acc_ref[...] = jnp.zeros((BQ, HD), F32)

    @pl.when(kt <= qt)
    def _():
        s = jax.lax.dot_general(q_ref[...], k_ref[...],
                                (((1,), (1,)), ((), ())),
                                preferred_element_type=F32) * SCALE
        row = qt * BQ + jax.lax.broadcasted_iota(jnp.int32, (BQ, BK), 0)
        col = kt * BK + jax.lax.broadcasted_iota(jnp.int32, (BQ, BK), 1)
        s = jnp.where(row >= col, s, NEG)
        m_prev = mx_ref[...]
        m_new = jnp.maximum(m_prev, jnp.max(s, axis=1, keepdims=True))
        corr = jnp.exp(m_prev - m_new)
        p = jnp.exp(s - m_new)
        ls_ref[...] = ls_ref[...] * corr + jnp.sum(p, axis=1, keepdims=True)
        pv = jnp.dot(p.astype(BF16), v_ref[...], preferred_element_type=F32)
        acc_ref[...] = acc_ref[...] * corr + pv
        mx_ref[...] = m_new

    @pl.when(kt == qt)
    def _():
        o_ref[...] = (acc_ref[...] / ls_ref[...]).astype(BF16)


def _attention(q, k, v):
    return pl.pallas_call(
        _attn_body,
        grid=(H, NQT, NKT),
        in_specs=[
            pl.BlockSpec((BQ, HD), lambda h, t, s: (t, h)),
            pl.BlockSpec((BK, HD), lambda h, t, s: (s, h)),
            pl.BlockSpec((BK, HD), lambda h, t, s: (s, h)),
        ],
        out_specs=pl.BlockSpec((BQ, HD), lambda h, t, s: (t, h)),
        out_shape=jax.ShapeDtypeStruct((T, D), BF16),
        scratch_shapes=[
            pltpu.VMEM((BQ, HD), F32),
            pltpu.VMEM((BQ, 1), F32),
            pltpu.VMEM((BQ, 1), F32),
        ],
        compiler_params=pltpu.CompilerParams(
            dimension_semantics=("arbitrary", "arbitrary", "arbitrary"),
        ),
    )(q, k, v)


def _onorm_body(a_ref, ow_ref, z_ref, w2_ref, z2_ref, h2_ref, owb_ref):
    t = pl.program_id(0)

    @pl.when(t == 0)
    def _():
        owb_ref[...] = ow_ref[...].astype(BF16)

    ao = jnp.dot(a_ref[...], owb_ref[...], preferred_element_type=F32)
    z2 = ao + z_ref[...]
    z2_ref[...] = z2
    v = jnp.mean(z2 * z2, axis=1, keepdims=True)
    h2_ref[...] = z2 * jax.lax.rsqrt(v + EPS) * w2_ref[...]


def _onorm(attn, o_w, z, ln2_row):
    tspec = pl.BlockSpec((BT, D), lambda t: (t, 0))
    return pl.pallas_call(
        _onorm_body,
        grid=(NT,),
        in_specs=[
            tspec,
            pl.BlockSpec((D, D), lambda t: (0, 0)),
            tspec,
            pl.BlockSpec((1, D), lambda t: (0, 0)),
        ],
        out_specs=[tspec, tspec],
        out_shape=[
            jax.ShapeDtypeStruct((T, D), F32),
            jax.ShapeDtypeStruct((T, D), F32),
        ],
        scratch_shapes=[pltpu.VMEM((D, D), BF16)],
        compiler_params=pltpu.CompilerParams(vmem_limit_bytes=100 * 2**20),
    )(attn, o_w, z, ln2_row)


def _sc_gather(table, idx, B, rpc):
    b_per_w = B // SC_NW
    chunks = b_per_w // rpc
    mesh = plsc.VectorSubcoreMesh(core_axis_name="c", subcore_axis_name="s")

    @functools.partial(
        pl.kernel,
        mesh=mesh,
        out_type=jax.ShapeDtypeStruct((B, D), F32),
        scratch_types=[
            pltpu.VMEM((rpc,), jnp.int32),
            pltpu.VMEM((rpc, D), F32),
            pltpu.SemaphoreType.DMA,
        ],
    )
    def k(table_hbm, idx_hbm, out_hbm, idx_v, rows_v, sem):
        wid = lax.axis_index("s") * SC_NC + lax.axis_index("c")
        for c in range(chunks):
            base = wid * b_per_w + c * rpc
            pltpu.sync_copy(idx_hbm.at[pl.ds(base, rpc)], idx_v)
            pltpu.async_copy(table_hbm.at[idx_v], rows_v, sem).wait()
            pltpu.sync_copy(rows_v, out_hbm.at[pl.ds(base, rpc)])

    return k(table, idx)


def _sc_scatter(src, dst_idx, B, rpc):
    b_per_w = T // SC_NW
    chunks = b_per_w // rpc
    mesh = plsc.VectorSubcoreMesh(core_axis_name="c", subcore_axis_name="s")

    @functools.partial(
        pl.kernel,
        mesh=mesh,
        out_type=jax.ShapeDtypeStruct((B, D), F32),
        scratch_types=[
            pltpu.VMEM((rpc,), jnp.int32),
            pltpu.VMEM((rpc, D), F32),
            pltpu.SemaphoreType.DMA,
        ],
    )
    def k(src_hbm, idx_hbm, out_hbm, idx_v, rows_v, sem):
        wid = lax.axis_index("s") * SC_NC + lax.axis_index("c")
        for c in range(chunks):
            base = wid * b_per_w + c * rpc
            pltpu.sync_copy(idx_hbm.at[pl.ds(base, rpc)], idx_v)
            pltpu.sync_copy(src_hbm.at[pl.ds(base, rpc)], rows_v)
            pltpu.async_copy(rows_v, out_hbm.at[idx_v], sem).wait()

    return k(src, dst_idx)


def _routing(gen_token_mask):
    m = gen_token_mask.astype(jnp.int32)
    cu = T - jnp.sum(m)
    ru = jnp.cumsum(1 - m)
    rg = jnp.cumsum(m)
    goff = ((cu + BTM - 1) // BTM) * BTM
    dest = jnp.where(m == 0, ru - 1, goff + rg - 1).astype(jnp.int32)
    e = (jnp.arange(NTM, dtype=jnp.int32) * BTM >= goff).astype(jnp.int32)
    return dest, e


def _cast_body(x_ref, o_ref):
    o_ref[...] = x_ref[...].astype(BF16)


def _cast_bf16(x):
    return pl.pallas_call(
        _cast_body,
        grid=(NTM,),
        in_specs=[pl.BlockSpec((BTM, D), lambda t: (t, 0))],
        out_specs=pl.BlockSpec((BTM, D), lambda t: (t, 0)),
        out_shape=jax.ShapeDtypeStruct((TP, D), BF16),
    )(x)


def _moe_body(e_ref, x_ref, guw_ref, uuw_ref, duw_ref, ggw_ref, ugw_ref,
              dgw_ref, out_ref, wu_ref, wg_ref, du_ref, dg_ref):
    j = pl.program_id(0)
    t = pl.program_id(1)

    @pl.when(t == 0)
    def _():
        valid = jnp.where(j == NJ2 - 1, I - (NJ2 - 1) * WI, WI)
        cm = jax.lax.broadcasted_iota(jnp.int32, (D, WI), 1) < valid
        rm = jax.lax.broadcasted_iota(jnp.int32, (WI, D), 0) < valid
        wu_ref[:, :WI] = jnp.where(cm, guw_ref[...], 0.0).astype(BF16)
        wu_ref[:, WI:] = jnp.where(cm, uuw_ref[...], 0.0).astype(BF16)
        wg_ref[:, :WI] = jnp.where(cm, ggw_ref[...], 0.0).astype(BF16)
        wg_ref[:, WI:] = jnp.where(cm, ugw_ref[...], 0.0).astype(BF16)
        du_ref[...] = jnp.where(rm, duw_ref[...], 0.0).astype(BF16)
        dg_ref[...] = jnp.where(rm, dgw_ref[...], 0.0).astype(BF16)

    x = x_ref[...]
    off = pl.multiple_of(t * BTM, BTM)

    def run(w_ref, d_ref):
        gu = jnp.dot(x, w_ref[...], preferred_element_type=F32)
        a = (_silu(gu[:, :WI]) * gu[:, WI:]).astype(BF16)
        o = jnp.dot(a, d_ref[...], preferred_element_type=F32)

        @pl.when(j == 0)
        def _():
            out_ref[pl.ds(off, BTM), :] = o

        @pl.when(j > 0)
        def _():
            out_ref[pl.ds(off, BTM), :] += o

    et = e_ref[t]

    @pl.when(et == 0)
    def _():
        run(wu_ref, du_ref)

    @pl.when(et != 0)
    def _():
        run(wg_ref, dg_ref)


def _moe(e, xp16, gate_w, up_w, down_w, gen_gate_w, gen_up_w, gen_down_w):
    xspec = pl.BlockSpec((BTM, D), lambda j, t, e_ref: (t, 0))
    gspec = pl.BlockSpec((D, WI), lambda j, t, e_ref: (0, j))
    dspec = pl.BlockSpec((WI, D), lambda j, t, e_ref: (j, 0))
    grid_spec = pltpu.PrefetchScalarGridSpec(
        num_scalar_prefetch=1,
        grid=(NJ2, NTM),
        in_specs=[xspec, gspec, gspec, dspec, gspec, gspec, dspec],
        out_specs=pl.BlockSpec((TP, D), lambda j, t, e_ref: (0, 0)),
        scratch_shapes=[
            pltpu.VMEM((D, 2 * WI), BF16),
            pltpu.VMEM((D, 2 * WI), BF16),
            pltpu.VMEM((WI, D), BF16),
            pltpu.VMEM((WI, D), BF16),
        ],
    )
    return pl.pallas_call(
        _moe_body,
        grid_spec=grid_spec,
        out_shape=jax.ShapeDtypeStruct((TP, D), F32),
        compiler_params=pltpu.CompilerParams(
            dimension_semantics=("arbitrary", "arbitrary"),
            vmem_limit_bytes=100 * 2**20,
        ),
    )(e, xp16, gate_w, up_w, down_w, gen_gate_w, gen_up_w, gen_down_w)


def kernel(positions, hidden_states, residual, gen_token_mask, ln1_w, ln2_w,
           q_w, q_b, k_w, k_b, v_w, v_b, o_w, gate_w, up_w, down_w,
           gen_gate_w, gen_up_w, gen_down_w):
    pos_col = positions.astype(F32).reshape(T, 1)
    ln1_row = ln1_w.reshape(1, D)
    ln2_row = ln2_w.reshape(1, D)
    qb3 = q_b.reshape(H // HG, 1, WQ)
    kb3 = k_b.reshape(H // HG, 1, WQ)
    vb3 = v_b.reshape(H // HG, 1, WQ)

    dest, e = _routing(gen_token_mask)
    z, xn = _addnorm1(hidden_states, residual, ln1_row)
    q, k, v = _qkv(xn, pos_col, q_w, k_w, v_w, qb3, kb3, vb3)
    attn = _attention(q, k, v)
    z2, h2 = _onorm(attn, o_w, z, ln2_row)
    xp = _sc_scatter(h2, dest, TP, 32)
    xp16 = _cast_bf16(xp)
    mo = _moe(e, xp16, gate_w, up_w, down_w, gen_gate_w, gen_up_w, gen_down_w)
    out = _sc_gather(mo, dest, T, 32)
    return (out, z2)


# speedup vs baseline: 1.2108x; 1.2108x over previous
import functools

import jax
import jax.numpy as jnp
from jax import lax
from jax.experimental import pallas as pl
from jax.experimental.pallas import tpu as pltpu
from jax.experimental.pallas import tpu_sc as plsc

EPS = 1e-6
THETA = 1000000.0
H = 16
HD = 128
T = 2048
D = 2048
I = 5504
BT = 256
NT = T // BT
WI = 256
NJ2 = (I + WI - 1) // WI
BTM = 512
NTM = T // BTM + 1
TP = NTM * BTM
SC_NC = 2
SC_NS = 16
SC_NW = SC_NC * SC_NS
SCALE = 1.0 / (HD ** 0.5)
NEG = -1e30
F32 = jnp.float32
BF16 = jnp.bfloat16


def _silu(x):
    return x / (1.0 + jnp.exp(-x))


def _addnorm1_body(h_ref, r_ref, w_ref, z_ref, xn_ref):
    z = h_ref[...] + r_ref[...]
    z_ref[...] = z
    v = jnp.mean(z * z, axis=1, keepdims=True)
    xn_ref[...] = (z * jax.lax.rsqrt(v + EPS) * w_ref[...]).astype(BF16)


def _addnorm1(h, r, ln1_row):
    return pl.pallas_call(
        _addnorm1_body,
        grid=(NT,),
        in_specs=[
            pl.BlockSpec((BT, D), lambda t: (t, 0)),
            pl.BlockSpec((BT, D), lambda t: (t, 0)),
            pl.BlockSpec((1, D), lambda t: (0, 0)),
        ],
        out_specs=[
            pl.BlockSpec((BT, D), lambda t: (t, 0)),
            pl.BlockSpec((BT, D), lambda t: (t, 0)),
        ],
        out_shape=[
            jax.ShapeDtypeStruct((T, D), F32),
            jax.ShapeDtypeStruct((T, D), BF16),
        ],
    )(h, r, ln1_row)


HG = 2
WQ = HG * HD


def _qkv_body(xn_ref, pos_ref, qw_ref, kw_ref, vw_ref, qb_ref, kb_ref, vb_ref,
              q_ref, k_ref, v_ref, cos_ref, sin_ref):
    j = pl.program_id(0)

    @pl.when(j == 0)
    def _():
        li = jax.lax.broadcasted_iota(jnp.int32, (T, WQ), 1)
        lh = li % HD
        f = jnp.where(lh < HD // 2, lh, lh - HD // 2).astype(F32)
        inv = jnp.exp(f * (-jnp.log(THETA) / (HD // 2)))
        freqs = pos_ref[...] * inv
        cos_ref[...] = jnp.cos(freqs)
        sgn = jnp.where(lh < HD // 2, -1.0, 1.0)
        sin_ref[...] = jnp.sin(freqs) * sgn

    xn = xn_ref[...]

    def proj(w_ref, b_ref):
        w = w_ref[...].astype(BF16)
        y = jnp.dot(xn, w, preferred_element_type=F32)
        return y + b_ref[0]

    def rope(x):
        x3 = x.reshape(T, HG, HD)
        xr = jnp.concatenate([x3[..., HD // 2:], x3[..., :HD // 2]],
                             axis=-1).reshape(T, WQ)
        return x * cos_ref[...] + xr * sin_ref[...]

    q_ref[...] = rope(proj(qw_ref, qb_ref)).astype(BF16)
    k_ref[...] = rope(proj(kw_ref, kb_ref)).astype(BF16)
    v_ref[...] = proj(vw_ref, vb_ref).astype(BF16)


def _qkv(xn, pos_col, q_w, k_w, v_w, qb3, kb3, vb3):
    wspec = pl.BlockSpec((D, WQ), lambda j: (0, j))
    bspec = pl.BlockSpec((1, 1, WQ), lambda j: (j, 0, 0))
    ospec = pl.BlockSpec((T, WQ), lambda j: (0, j))
    return pl.pallas_call(
        _qkv_body,
        grid=(H // HG,),
        in_specs=[
            pl.BlockSpec((T, D), lambda j: (0, 0)),
            pl.BlockSpec((T, 1), lambda j: (0, 0)),
            wspec, wspec, wspec, bspec, bspec, bspec,
        ],
        out_specs=[ospec, ospec, ospec],
        out_shape=[jax.ShapeDtypeStruct((T, D), BF16)] * 3,
        scratch_shapes=[
            pltpu.VMEM((T, WQ), F32),
            pltpu.VMEM((T, WQ), F32),
        ],
        compiler_params=pltpu.CompilerParams(vmem_limit_bytes=100 * 2**20),
    )(xn, pos_col, q_w, k_w, v_w, qb3, kb3, vb3)


def _attn_body(q_ref, k_ref, v_ref, o_ref):
    qt = pl.program_id(1)
    q = q_ref[...]
    s = jax.lax.dot_general(q, k_ref[...], (((1,), (1,)), ((), ())),
                            preferred_element_type=F32) * SCALE
    row = qt * BT + jax.lax.broadcasted_iota(jnp.int32, (BT, T), 0)
    col = jax.lax.broadcasted_iota(jnp.int32, (BT, T), 1)
    s = jnp.where(row >= col, s, NEG)
    m = jnp.max(s, axis=1, keepdims=True)
    p = jnp.exp(s - m)
    p = p / jnp.sum(p, axis=1, keepdims=True)
    o = jnp.dot(p.astype(BF16), v_ref[...], preferred_element_type=F32)
    o_ref[...] = o.astype(BF16)


def _attention(q, k, v):
    return pl.pallas_call(
        _attn_body,
        grid=(H, NT),
        in_specs=[
            pl.BlockSpec((BT, HD), lambda h, t: (t, h)),
            pl.BlockSpec((T, HD), lambda h, t: (0, h)),
            pl.BlockSpec((T, HD), lambda h, t: (0, h)),
        ],
        out_specs=pl.BlockSpec((BT, HD), lambda h, t: (t, h)),
        out_shape=jax.ShapeDtypeStruct((T, D), BF16),
    )(q, k, v)


def _onorm_body(a_ref, ow_ref, z_ref, w2_ref, z2_ref, h2_ref, owb_ref):
    t = pl.program_id(0)

    @pl.when(t == 0)
    def _():
        owb_ref[...] = ow_ref[...].astype(BF16)

    ao = jnp.dot(a_ref[...], owb_ref[...], preferred_element_type=F32)
    z2 = ao + z_ref[...]
    z2_ref[...] = z2
    v = jnp.mean(z2 * z2, axis=1, keepdims=True)
    h2_ref[...] = (z2 * jax.lax.rsqrt(v + EPS) * w2_ref[...]).astype(BF16)


def _onorm(attn, o_w, z, ln2_row):
    tspec = pl.BlockSpec((BT, D), lambda t: (t, 0))
    return pl.pallas_call(
        _onorm_body,
        grid=(NT,),
        in_specs=[
            tspec,
            pl.BlockSpec((D, D), lambda t: (0, 0)),
            tspec,
            pl.BlockSpec((1, D), lambda t: (0, 0)),
        ],
        out_specs=[tspec, tspec],
        out_shape=[
            jax.ShapeDtypeStruct((T, D), F32),
            jax.ShapeDtypeStruct((T, D), BF16),
        ],
        scratch_shapes=[pltpu.VMEM((D, D), BF16)],
        compiler_params=pltpu.CompilerParams(vmem_limit_bytes=100 * 2**20),
    )(attn, o_w, z, ln2_row)


def _sc_gather(table, idx, B, rpc):
    b_per_w = B // SC_NW
    chunks = b_per_w // rpc
    mesh = plsc.VectorSubcoreMesh(core_axis_name="c", subcore_axis_name="s")

    @functools.partial(
        pl.kernel,
        mesh=mesh,
        out_type=jax.ShapeDtypeStruct((B, D), F32),
        scratch_types=[
            pltpu.VMEM((rpc,), jnp.int32),
            pltpu.VMEM((rpc, D), F32),
            pltpu.SemaphoreType.DMA,
        ],
    )
    def k(table_hbm, idx_hbm, out_hbm, idx_v, rows_v, sem):
        wid = lax.axis_index("s") * SC_NC + lax.axis_index("c")
        for c in range(chunks):
            base = wid * b_per_w + c * rpc
            pltpu.sync_copy(idx_hbm.at[pl.ds(base, rpc)], idx_v)
            pltpu.async_copy(table_hbm.at[idx_v], rows_v, sem).wait()
            pltpu.sync_copy(rows_v, out_hbm.at[pl.ds(base, rpc)])

    return k(table, idx)


def _sc_scatter(src, dst_idx, B, rpc):
    b_per_w = T // SC_NW
    chunks = b_per_w // rpc
    mesh = plsc.VectorSubcoreMesh(core_axis_name="c", subcore_axis_name="s")

    @functools.partial(
        pl.kernel,
        mesh=mesh,
        out_type=jax.ShapeDtypeStruct((B, D), F32),
        scratch_types=[
            pltpu.VMEM((rpc,), jnp.int32),
            pltpu.VMEM((rpc, D), F32),
            pltpu.SemaphoreType.DMA,
        ],
    )
    def k(src_hbm, idx_hbm, out_hbm, idx_v, rows_v, sem):
        wid = lax.axis_index("s") * SC_NC + lax.axis_index("c")
        for c in range(chunks):
            base = wid * b_per_w + c * rpc
            pltpu.sync_copy(idx_hbm.at[pl.ds(base, rpc)], idx_v)
            pltpu.sync_copy(src_hbm.at[pl.ds(base, rpc)], rows_v)
            pltpu.async_copy(rows_v, out_hbm.at[idx_v], sem).wait()

    return k(src, dst_idx)


def _routing(gen_token_mask):
    m = gen_token_mask.astype(jnp.int32)
    cu = T - jnp.sum(m)
    ru = jnp.cumsum(1 - m)
    rg = jnp.cumsum(m)
    goff = ((cu + BTM - 1) // BTM) * BTM
    dest = jnp.where(m == 0, ru - 1, goff + rg - 1).astype(jnp.int32)
    e = (jnp.arange(NTM, dtype=jnp.int32) * BTM >= goff).astype(jnp.int32)
    return dest, e


WI2 = 128


def _moem_body(x_ref, mu_ref, mg_ref, guw_ref, uuw_ref, duw_ref, ggw_ref,
               ugw_ref, dgw_ref, out_ref, wu_ref, wg_ref, dc_ref, a_ref):
    j = pl.program_id(0)
    wu_ref[:, :WI2] = guw_ref[...].astype(BF16)
    wu_ref[:, WI2:] = uuw_ref[...].astype(BF16)
    wg_ref[:, :WI2] = ggw_ref[...].astype(BF16)
    wg_ref[:, WI2:] = ugw_ref[...].astype(BF16)
    dc_ref[:WI2, :] = duw_ref[...].astype(BF16)
    dc_ref[WI2:, :] = dgw_ref[...].astype(BF16)

    x = x_ref[...]
    guu = jnp.dot(x, wu_ref[...], preferred_element_type=F32)
    a_ref[:, :WI2] = (_silu(guu[:, :WI2]) * guu[:, WI2:]
                      * mu_ref[...]).astype(BF16)
    gug = jnp.dot(x, wg_ref[...], preferred_element_type=F32)
    a_ref[:, WI2:] = (_silu(gug[:, :WI2]) * gug[:, WI2:]
                      * mg_ref[...]).astype(BF16)
    o = jnp.dot(a_ref[...], dc_ref[...], preferred_element_type=F32)

    @pl.when(j == 0)
    def _():
        out_ref[...] = o

    @pl.when(j > 0)
    def _():
        out_ref[...] += o


def _moe_masked(x16, mu, mg, gate_w, up_w, down_w, gen_gate_w, gen_up_w,
                gen_down_w):
    cspec = pl.BlockSpec((T, D), lambda j: (0, 0))
    mspec = pl.BlockSpec((T, WI2), lambda j: (0, 0))
    gspec = pl.BlockSpec((D, WI2), lambda j: (0, j))
    dspec = pl.BlockSpec((WI2, D), lambda j: (j, 0))
    return pl.pallas_call(
        _moem_body,
        grid=(I // WI2,),
        in_specs=[cspec, mspec, mspec, gspec, gspec, dspec, gspec, gspec,
                  dspec],
        out_specs=pl.BlockSpec((T, D), lambda j: (0, 0)),
        out_shape=jax.ShapeDtypeStruct((T, D), F32),
        scratch_shapes=[
            pltpu.VMEM((D, 2 * WI2), BF16),
            pltpu.VMEM((D, 2 * WI2), BF16),
            pltpu.VMEM((2 * WI2, D), BF16),
            pltpu.VMEM((T, 2 * WI2), BF16),
        ],
        compiler_params=pltpu.CompilerParams(
            dimension_semantics=("arbitrary",),
            vmem_limit_bytes=100 * 2**20,
        ),
    )(x16, mu, mg, gate_w, up_w, down_w, gen_gate_w, gen_up_w, gen_down_w)


def _cast_body(x_ref, o_ref):
    o_ref[...] = x_ref[...].astype(BF16)


def _cast_bf16(x):
    return pl.pallas_call(
        _cast_body,
        grid=(NTM,),
        in_specs=[pl.BlockSpec((BTM, D), lambda t: (t, 0))],
        out_specs=pl.BlockSpec((BTM, D), lambda t: (t, 0)),
        out_shape=jax.ShapeDtypeStruct((TP, D), BF16),
    )(x)


def _moe_body(e_ref, x_ref, guw_ref, uuw_ref, duw_ref, ggw_ref, ugw_ref,
              dgw_ref, out_ref, wu_ref, wg_ref, du_ref, dg_ref):
    j = pl.program_id(0)
    t = pl.program_id(1)

    @pl.when(t == 0)
    def _():
        valid = jnp.where(j == NJ2 - 1, I - (NJ2 - 1) * WI, WI)
        cm = jax.lax.broadcasted_iota(jnp.int32, (D, WI), 1) < valid
        rm = jax.lax.broadcasted_iota(jnp.int32, (WI, D), 0) < valid
        wu_ref[:, :WI] = jnp.where(cm, guw_ref[...], 0.0).astype(BF16)
        wu_ref[:, WI:] = jnp.where(cm, uuw_ref[...], 0.0).astype(BF16)
        wg_ref[:, :WI] = jnp.where(cm, ggw_ref[...], 0.0).astype(BF16)
        wg_ref[:, WI:] = jnp.where(cm, ugw_ref[...], 0.0).astype(BF16)
        du_ref[...] = jnp.where(rm, duw_ref[...], 0.0).astype(BF16)
        dg_ref[...] = jnp.where(rm, dgw_ref[...], 0.0).astype(BF16)

    x = x_ref[...]
    off = pl.multiple_of(t * BTM, BTM)

    def run(w_ref, d_ref):
        gu = jnp.dot(x, w_ref[...], preferred_element_type=F32)
        a = (_silu(gu[:, :WI]) * gu[:, WI:]).astype(BF16)
        o = jnp.dot(a, d_ref[...], preferred_element_type=F32)

        @pl.when(j == 0)
        def _():
            out_ref[pl.ds(off, BTM), :] = o

        @pl.when(j > 0)
        def _():
            out_ref[pl.ds(off, BTM), :] += o

    et = e_ref[t]

    @pl.when(et == 0)
    def _():
        run(wu_ref, du_ref)

    @pl.when(et != 0)
    def _():
        run(wg_ref, dg_ref)


def _moe(e, xp16, gate_w, up_w, down_w, gen_gate_w, gen_up_w, gen_down_w):
    xspec = pl.BlockSpec((BTM, D), lambda j, t, e_ref: (t, 0))
    gspec = pl.BlockSpec((D, WI), lambda j, t, e_ref: (0, j))
    dspec = pl.BlockSpec((WI, D), lambda j, t, e_ref: (j, 0))
    grid_spec = pltpu.PrefetchScalarGridSpec(
        num_scalar_prefetch=1,
        grid=(NJ2, NTM),
        in_specs=[xspec, gspec, gspec, dspec, gspec, gspec, dspec],
        out_specs=pl.BlockSpec((TP, D), lambda j, t, e_ref: (0, 0)),
        scratch_shapes=[
            pltpu.VMEM((D, 2 * WI), BF16),
            pltpu.VMEM((D, 2 * WI), BF16),
            pltpu.VMEM((WI, D), BF16),
            pltpu.VMEM((WI, D), BF16),
        ],
    )
    return pl.pallas_call(
        _moe_body,
        grid_spec=grid_spec,
        out_shape=jax.ShapeDtypeStruct((TP, D), F32),
        compiler_params=pltpu.CompilerParams(
            dimension_semantics=("arbitrary", "arbitrary"),
            vmem_limit_bytes=100 * 2**20,
        ),
    )(e, xp16, gate_w, up_w, down_w, gen_gate_w, gen_up_w, gen_down_w)


def kernel(positions, hidden_states, residual, gen_token_mask, ln1_w, ln2_w,
           q_w, q_b, k_w, k_b, v_w, v_b, o_w, gate_w, up_w, down_w,
           gen_gate_w, gen_up_w, gen_down_w):
    pos_col = positions.astype(F32).reshape(T, 1)
    ln1_row = ln1_w.reshape(1, D)
    ln2_row = ln2_w.reshape(1, D)
    qb3 = q_b.reshape(H // HG, 1, WQ)
    kb3 = k_b.reshape(H // HG, 1, WQ)
    vb3 = v_b.reshape(H // HG, 1, WQ)

    mg_col = gen_token_mask.astype(F32).reshape(T, 1)
    mg = jnp.broadcast_to(mg_col, (T, WI2))
    mu = 1.0 - mg

    z, xn = _addnorm1(hidden_states, residual, ln1_row)
    q, k, v = _qkv(xn, pos_col, q_w, k_w, v_w, qb3, kb3, vb3)
    attn = _attention(q, k, v)
    z2, h2 = _onorm(attn, o_w, z, ln2_row)
    out = _moe_masked(h2, mu, mg, gate_w, up_w, down_w, gen_gate_w,
                      gen_up_w, gen_down_w)
    return (out, z2)


# speedup vs baseline: 1.2206x; 1.0081x over previous
import functools

import jax
import jax.numpy as jnp
from jax import lax
from jax.experimental import pallas as pl
from jax.experimental.pallas import tpu as pltpu
from jax.experimental.pallas import tpu_sc as plsc

EPS = 1e-6
THETA = 1000000.0
H = 16
HD = 128
T = 2048
D = 2048
I = 5504
BT = 256
NT = T // BT
WI = 256
NJ2 = (I + WI - 1) // WI
BTM = 512
NTM = T // BTM + 1
TP = NTM * BTM
SC_NC = 2
SC_NS = 16
SC_NW = SC_NC * SC_NS
SCALE = 1.0 / (HD ** 0.5)
NEG = -1e30
F32 = jnp.float32
BF16 = jnp.bfloat16


def _silu(x):
    return x / (1.0 + jnp.exp(-x))


def _addnorm1_body(h_ref, r_ref, w_ref, z_ref, xn_ref):
    z = h_ref[...] + r_ref[...]
    z_ref[...] = z
    v = jnp.mean(z * z, axis=1, keepdims=True)
    xn_ref[...] = (z * jax.lax.rsqrt(v + EPS) * w_ref[...]).astype(BF16)


def _addnorm1(h, r, ln1_row):
    return pl.pallas_call(
        _addnorm1_body,
        grid=(NT,),
        in_specs=[
            pl.BlockSpec((BT, D), lambda t: (t, 0)),
            pl.BlockSpec((BT, D), lambda t: (t, 0)),
            pl.BlockSpec((1, D), lambda t: (0, 0)),
        ],
        out_specs=[
            pl.BlockSpec((BT, D), lambda t: (t, 0)),
            pl.BlockSpec((BT, D), lambda t: (t, 0)),
        ],
        out_shape=[
            jax.ShapeDtypeStruct((T, D), F32),
            jax.ShapeDtypeStruct((T, D), BF16),
        ],
    )(h, r, ln1_row)


HG = 2
WQ = HG * HD


def _qkv_body(xn_ref, pos_ref, qw_ref, kw_ref, vw_ref, qb_ref, kb_ref, vb_ref,
              q_ref, k_ref, v_ref, cos_ref, sin_ref):
    j = pl.program_id(0)

    @pl.when(j == 0)
    def _():
        li = jax.lax.broadcasted_iota(jnp.int32, (T, WQ), 1)
        lh = li % HD
        f = jnp.where(lh < HD // 2, lh, lh - HD // 2).astype(F32)
        inv = jnp.exp(f * (-jnp.log(THETA) / (HD // 2)))
        freqs = pos_ref[...] * inv
        cos_ref[...] = jnp.cos(freqs)
        sgn = jnp.where(lh < HD // 2, -1.0, 1.0)
        sin_ref[...] = jnp.sin(freqs) * sgn

    xn = xn_ref[...]

    def proj(w_ref, b_ref):
        w = w_ref[...].astype(BF16)
        y = jnp.dot(xn, w, preferred_element_type=F32)
        return y + b_ref[0]

    def rope(x):
        x3 = x.reshape(T, HG, HD)
        xr = jnp.concatenate([x3[..., HD // 2:], x3[..., :HD // 2]],
                             axis=-1).reshape(T, WQ)
        return x * cos_ref[...] + xr * sin_ref[...]

    q_ref[...] = rope(proj(qw_ref, qb_ref)).astype(BF16)
    k_ref[...] = rope(proj(kw_ref, kb_ref)).astype(BF16)
    v_ref[...] = proj(vw_ref, vb_ref).astype(BF16)


def _qkv(xn, pos_col, q_w, k_w, v_w, qb3, kb3, vb3):
    wspec = pl.BlockSpec((D, WQ), lambda j: (0, j))
    bspec = pl.BlockSpec((1, 1, WQ), lambda j: (j, 0, 0))
    ospec = pl.BlockSpec((T, WQ), lambda j: (0, j))
    return pl.pallas_call(
        _qkv_body,
        grid=(H // HG,),
        in_specs=[
            pl.BlockSpec((T, D), lambda j: (0, 0)),
            pl.BlockSpec((T, 1), lambda j: (0, 0)),
            wspec, wspec, wspec, bspec, bspec, bspec,
        ],
        out_specs=[ospec, ospec, ospec],
        out_shape=[jax.ShapeDtypeStruct((T, D), BF16)] * 3,
        scratch_shapes=[
            pltpu.VMEM((T, WQ), F32),
            pltpu.VMEM((T, WQ), F32),
        ],
        compiler_params=pltpu.CompilerParams(vmem_limit_bytes=100 * 2**20),
    )(xn, pos_col, q_w, k_w, v_w, qb3, kb3, vb3)


def _attn_body(q_ref, k_ref, v_ref, o_ref):
    qt = pl.program_id(1)
    q = q_ref[...]
    s = jax.lax.dot_general(q, k_ref[...], (((1,), (1,)), ((), ())),
                            preferred_element_type=F32) * SCALE
    row = qt * BT + jax.lax.broadcasted_iota(jnp.int32, (BT, T), 0)
    col = jax.lax.broadcasted_iota(jnp.int32, (BT, T), 1)
    s = jnp.where(row >= col, s, NEG)
    m = jnp.max(s, axis=1, keepdims=True)
    p = jnp.exp(s - m)
    p = p / jnp.sum(p, axis=1, keepdims=True)
    o = jnp.dot(p.astype(BF16), v_ref[...], preferred_element_type=F32)
    o_ref[...] = o.astype(BF16)


def _attention(q, k, v):
    return pl.pallas_call(
        _attn_body,
        grid=(H, NT),
        in_specs=[
            pl.BlockSpec((BT, HD), lambda h, t: (t, h)),
            pl.BlockSpec((T, HD), lambda h, t: (0, h)),
            pl.BlockSpec((T, HD), lambda h, t: (0, h)),
        ],
        out_specs=pl.BlockSpec((BT, HD), lambda h, t: (t, h)),
        out_shape=jax.ShapeDtypeStruct((T, D), BF16),
    )(q, k, v)


def _onorm_body(a_ref, ow_ref, z_ref, w2_ref, z2_ref, h2_ref, owb_ref):
    t = pl.program_id(0)

    @pl.when(t == 0)
    def _():
        owb_ref[...] = ow_ref[...].astype(BF16)

    ao = jnp.dot(a_ref[...], owb_ref[...], preferred_element_type=F32)
    z2 = ao + z_ref[...]
    z2_ref[...] = z2
    v = jnp.mean(z2 * z2, axis=1, keepdims=True)
    h2_ref[...] = (z2 * jax.lax.rsqrt(v + EPS) * w2_ref[...]).astype(BF16)


def _onorm(attn, o_w, z, ln2_row):
    tspec = pl.BlockSpec((BT, D), lambda t: (t, 0))
    return pl.pallas_call(
        _onorm_body,
        grid=(NT,),
        in_specs=[
            tspec,
            pl.BlockSpec((D, D), lambda t: (0, 0)),
            tspec,
            pl.BlockSpec((1, D), lambda t: (0, 0)),
        ],
        out_specs=[tspec, tspec],
        out_shape=[
            jax.ShapeDtypeStruct((T, D), F32),
            jax.ShapeDtypeStruct((T, D), BF16),
        ],
        scratch_shapes=[pltpu.VMEM((D, D), BF16)],
        compiler_params=pltpu.CompilerParams(vmem_limit_bytes=100 * 2**20),
    )(attn, o_w, z, ln2_row)


def _sc_gather(table, idx, B, rpc):
    b_per_w = B // SC_NW
    chunks = b_per_w // rpc
    mesh = plsc.VectorSubcoreMesh(core_axis_name="c", subcore_axis_name="s")

    @functools.partial(
        pl.kernel,
        mesh=mesh,
        out_type=jax.ShapeDtypeStruct((B, D), F32),
        scratch_types=[
            pltpu.VMEM((rpc,), jnp.int32),
            pltpu.VMEM((rpc, D), F32),
            pltpu.SemaphoreType.DMA,
        ],
    )
    def k(table_hbm, idx_hbm, out_hbm, idx_v, rows_v, sem):
        wid = lax.axis_index("s") * SC_NC + lax.axis_index("c")
        for c in range(chunks):
            base = wid * b_per_w + c * rpc
            pltpu.sync_copy(idx_hbm.at[pl.ds(base, rpc)], idx_v)
            pltpu.async_copy(table_hbm.at[idx_v], rows_v, sem).wait()
            pltpu.sync_copy(rows_v, out_hbm.at[pl.ds(base, rpc)])

    return k(table, idx)


def _sc_scatter(src, dst_idx, B, rpc):
    b_per_w = T // SC_NW
    chunks = b_per_w // rpc
    mesh = plsc.VectorSubcoreMesh(core_axis_name="c", subcore_axis_name="s")

    @functools.partial(
        pl.kernel,
        mesh=mesh,
        out_type=jax.ShapeDtypeStruct((B, D), F32),
        scratch_types=[
            pltpu.VMEM((rpc,), jnp.int32),
            pltpu.VMEM((rpc, D), F32),
            pltpu.SemaphoreType.DMA,
        ],
    )
    def k(src_hbm, idx_hbm, out_hbm, idx_v, rows_v, sem):
        wid = lax.axis_index("s") * SC_NC + lax.axis_index("c")
        for c in range(chunks):
            base = wid * b_per_w + c * rpc
            pltpu.sync_copy(idx_hbm.at[pl.ds(base, rpc)], idx_v)
            pltpu.sync_copy(src_hbm.at[pl.ds(base, rpc)], rows_v)
            pltpu.async_copy(rows_v, out_hbm.at[idx_v], sem).wait()

    return k(src, dst_idx)


def _routing(gen_token_mask):
    m = gen_token_mask.astype(jnp.int32)
    cu = T - jnp.sum(m)
    ru = jnp.cumsum(1 - m)
    rg = jnp.cumsum(m)
    goff = ((cu + BTM - 1) // BTM) * BTM
    dest = jnp.where(m == 0, ru - 1, goff + rg - 1).astype(jnp.int32)
    e = (jnp.arange(NTM, dtype=jnp.int32) * BTM >= goff).astype(jnp.int32)
    return dest, e


WI2 = 128
NJB = I // WI2


def _moem_body(x_ref, mu_ref, mg_ref, guw_ref, uuw_ref, duw_ref, ggw_ref,
               ugw_ref, dgw_ref, out_ref, wu_ref, wg_ref, dc_ref, a_ref):
    j = pl.program_id(0)
    wu_ref[:, :WI2] = guw_ref[...].astype(BF16)
    wu_ref[:, WI2:] = uuw_ref[...].astype(BF16)
    wg_ref[:, :WI2] = ggw_ref[...].astype(BF16)
    wg_ref[:, WI2:] = ugw_ref[...].astype(BF16)
    dc_ref[:WI2, :] = duw_ref[...].astype(BF16)
    dc_ref[WI2:, :] = dgw_ref[...].astype(BF16)

    x = x_ref[...]
    guu = jnp.dot(x, wu_ref[...], preferred_element_type=F32)
    a_ref[:, :WI2] = (_silu(guu[:, :WI2]) * guu[:, WI2:]
                      * mu_ref[...]).astype(BF16)
    gug = jnp.dot(x, wg_ref[...], preferred_element_type=F32)
    a_ref[:, WI2:] = (_silu(gug[:, :WI2]) * gug[:, WI2:]
                      * mg_ref[...]).astype(BF16)
    o = jnp.dot(a_ref[...], dc_ref[...], preferred_element_type=F32)

    @pl.when(j == 0)
    def _():
        out_ref[...] = o

    @pl.when(j > 0)
    def _():
        out_ref[...] += o


def _moe_masked(x16, mu, mg, gate_w, up_w, down_w, gen_gate_w, gen_up_w,
                gen_down_w):
    cspec = pl.BlockSpec((T, D), lambda j: (0, 0))
    mspec = pl.BlockSpec((T, WI2), lambda j: (0, 0))
    gspec = pl.BlockSpec((D, WI2), lambda j: (0, j))
    dspec = pl.BlockSpec((WI2, D), lambda j: (j, 0))
    return pl.pallas_call(
        _moem_body,
        grid=(NJB,),
        in_specs=[cspec, mspec, mspec, gspec, gspec, dspec, gspec, gspec,
                  dspec],
        out_specs=pl.BlockSpec((T, D), lambda j: (0, 0)),
        out_shape=jax.ShapeDtypeStruct((T, D), F32),
        scratch_shapes=[
            pltpu.VMEM((D, 2 * WI2), BF16),
            pltpu.VMEM((D, 2 * WI2), BF16),
            pltpu.VMEM((2 * WI2, D), BF16),
            pltpu.VMEM((T, 2 * WI2), BF16),
        ],
        compiler_params=pltpu.CompilerParams(
            dimension_semantics=("arbitrary",),
            vmem_limit_bytes=100 * 2**20,
        ),
    )(x16, mu, mg, gate_w, up_w, down_w, gen_gate_w, gen_up_w, gen_down_w)


def _cast_body(x_ref, o_ref):
    o_ref[...] = x_ref[...].astype(BF16)


def _cast_bf16(x):
    return pl.pallas_call(
        _cast_body,
        grid=(NTM,),
        in_specs=[pl.BlockSpec((BTM, D), lambda t: (t, 0))],
        out_specs=pl.BlockSpec((BTM, D), lambda t: (t, 0)),
        out_shape=jax.ShapeDtypeStruct((TP, D), BF16),
    )(x)


def _moe_body(e_ref, x_ref, guw_ref, uuw_ref, duw_ref, ggw_ref, ugw_ref,
              dgw_ref, out_ref, wu_ref, wg_ref, du_ref, dg_ref):
    j = pl.program_id(0)
    t = pl.program_id(1)

    @pl.when(t == 0)
    def _():
        valid = jnp.where(j == NJ2 - 1, I - (NJ2 - 1) * WI, WI)
        cm = jax.lax.broadcasted_iota(jnp.int32, (D, WI), 1) < valid
        rm = jax.lax.broadcasted_iota(jnp.int32, (WI, D), 0) < valid
        wu_ref[:, :WI] = jnp.where(cm, guw_ref[...], 0.0).astype(BF16)
        wu_ref[:, WI:] = jnp.where(cm, uuw_ref[...], 0.0).astype(BF16)
        wg_ref[:, :WI] = jnp.where(cm, ggw_ref[...], 0.0).astype(BF16)
        wg_ref[:, WI:] = jnp.where(cm, ugw_ref[...], 0.0).astype(BF16)
        du_ref[...] = jnp.where(rm, duw_ref[...], 0.0).astype(BF16)
        dg_ref[...] = jnp.where(rm, dgw_ref[...], 0.0).astype(BF16)

    x = x_ref[...]
    off = pl.multiple_of(t * BTM, BTM)

    def run(w_ref, d_ref):
        gu = jnp.dot(x, w_ref[...], preferred_element_type=F32)
        a = (_silu(gu[:, :WI]) * gu[:, WI:]).astype(BF16)
        o = jnp.dot(a, d_ref[...], preferred_element_type=F32)

        @pl.when(j == 0)
        def _():
            out_ref[pl.ds(off, BTM), :] = o

        @pl.when(j > 0)
        def _():
            out_ref[pl.ds(off, BTM), :] += o

    et = e_ref[t]

    @pl.when(et == 0)
    def _():
        run(wu_ref, du_ref)

    @pl.when(et != 0)
    def _():
        run(wg_ref, dg_ref)


def _moe(e, xp16, gate_w, up_w, down_w, gen_gate_w, gen_up_w, gen_down_w):
    xspec = pl.BlockSpec((BTM, D), lambda j, t, e_ref: (t, 0))
    gspec = pl.BlockSpec((D, WI), lambda j, t, e_ref: (0, j))
    dspec = pl.BlockSpec((WI, D), lambda j, t, e_ref: (j, 0))
    grid_spec = pltpu.PrefetchScalarGridSpec(
        num_scalar_prefetch=1,
        grid=(NJ2, NTM),
        in_specs=[xspec, gspec, gspec, dspec, gspec, gspec, dspec],
        out_specs=pl.BlockSpec((TP, D), lambda j, t, e_ref: (0, 0)),
        scratch_shapes=[
            pltpu.VMEM((D, 2 * WI), BF16),
            pltpu.VMEM((D, 2 * WI), BF16),
            pltpu.VMEM((WI, D), BF16),
            pltpu.VMEM((WI, D), BF16),
        ],
    )
    return pl.pallas_call(
        _moe_body,
        grid_spec=grid_spec,
        out_shape=jax.ShapeDtypeStruct((TP, D), F32),
        compiler_params=pltpu.CompilerParams(
            dimension_semantics=("arbitrary", "arbitrary"),
            vmem_limit_bytes=100 * 2**20,
        ),
    )(e, xp16, gate_w, up_w, down_w, gen_gate_w, gen_up_w, gen_down_w)


def kernel(positions, hidden_states, residual, gen_token_mask, ln1_w, ln2_w,
           q_w, q_b, k_w, k_b, v_w, v_b, o_w, gate_w, up_w, down_w,
           gen_gate_w, gen_up_w, gen_down_w):
    pos_col = positions.astype(F32).reshape(T, 1)
    ln1_row = ln1_w.reshape(1, D)
    ln2_row = ln2_w.reshape(1, D)
    qb3 = q_b.reshape(H // HG, 1, WQ)
    kb3 = k_b.reshape(H // HG, 1, WQ)
    vb3 = v_b.reshape(H // HG, 1, WQ)

    mg_col = gen_token_mask.astype(F32).reshape(T, 1)
    mg = jnp.broadcast_to(mg_col, (T, WI2))
    mu = 1.0 - mg

    z, xn = _addnorm1(hidden_states, residual, ln1_row)
    q, k, v = _qkv(xn, pos_col, q_w, k_w, v_w, qb3, kb3, vb3)
    attn = _attention(q, k, v)
    z2, h2 = _onorm(attn, o_w, z, ln2_row)
    out = _moe_masked(h2, mu, mg, gate_w, up_w, down_w, gen_gate_w,
                      gen_up_w, gen_down_w)
    return (out, z2)
